# baseline (speedup 1.0000x reference)
import functools

import jax
import jax.numpy as jnp
import numpy as np
from jax import lax
from jax.experimental import pallas as pl
from jax.experimental.pallas import tpu as pltpu
from jax.experimental.pallas import tpu_sc as plsc

_EPS = 0.1
_G = 14


def _to_z_sc(n, p, hw, x_hbm, o_hbm, bufs, tailb, sems, tsem):
    info = plsc.get_sparse_core_info()
    nc, ns = info.num_cores, info.num_subcores
    nw = nc * ns
    nch = (1 + p) // _G
    per_w = n // nw

    wid = lax.axis_index("s") * nc + lax.axis_index("c")
    lanes = jnp.arange(16, dtype=jnp.int32)
    zeros16 = jnp.zeros((16,), jnp.float32)

    g2 = hw - 16

    def _zero_img(ref, g):
        def _zr(h, _):
            ref[g, 0, h, pl.ds(0, 16)] = zeros16
            ref[g, 0, h, pl.ds(g2, 16)] = zeros16
            return 0

        lax.fori_loop(0, hw, _zr, 0)

    def _zero_slot(g, _):
        for b in range(2):
            _zero_img(bufs.at[b], g)
        return 0

    lax.fori_loop(0, _G, _zero_slot, 0)
    _zero_img(tailb, 0)

    def _eps_store(ref, g, k, val16):
        h = (k - 1) // hw
        w = (k - 1) % hw

        @pl.when(w < 16)
        def _lo():
            ref[g, 0, h, pl.ds(0, 16)] = val16(w)

        @pl.when(w >= 16)
        def _hi():
            ref[g, 0, h, pl.ds(g2, 16)] = val16(w - g2)

    def _onehot(off):
        return jnp.where(lanes == off, _EPS, 0.0).astype(jnp.float32)

    def _zerov(off):
        return zeros16

    _eps_store(tailb, 0, jnp.int32(p), _onehot)

    def _chunk_dmas(b, c, start):
        for s in range(per_w):
            batch = wid * per_w + s
            cp = pltpu.make_async_copy(
                bufs.at[b],
                o_hbm.at[batch, pl.ds(c * _G, _G)],
                sems.at[b],
            )
            if start:
                cp.start()
            else:
                cp.wait()

    def _set_c0(g, _):
        _eps_store(bufs.at[0], g, g, _onehot)
        return 0

    lax.fori_loop(1, _G, _set_c0, 0)
    for s in range(per_w):
        batch = wid * per_w + s
        pltpu.sync_copy(x_hbm.at[batch], bufs.at[0, 0])
        pltpu.make_async_copy(
            bufs.at[0], o_hbm.at[batch, pl.ds(0, _G)], sems.at[0]
        ).start()
        pltpu.make_async_copy(
            bufs.at[0], o_hbm.at[batch, pl.ds(0, _G)], sems.at[0]
        ).wait()
        pltpu.make_async_copy(
            tailb, o_hbm.at[batch, pl.ds(nch * _G, 1)], tsem
        ).start()

    _zero_img(bufs.at[0], 0)

    def _clr_c0(g, _):
        _eps_store(bufs.at[0], g, g, _zerov)
        return 0

    lax.fori_loop(1, _G, _clr_c0, 0)

    def _do_chunk(b, c):
        @pl.when(c > 2)
        def _wait_prev():
            _chunk_dmas(b, c - 2, start=False)

        def _edit(g, _):
            k = c * _G + g
            k_old = k - 2 * _G

            @pl.when(k_old >= 1)
            def _clr():
                _eps_store(bufs.at[b], g, k_old, _zerov)

            _eps_store(bufs.at[b], g, k, _onehot)
            return 0

        lax.fori_loop(0, _G, _edit, 0)
        _chunk_dmas(b, c, start=True)

    def _pair(t, _):
        for b in range(2):
            _do_chunk(b, 2 * t + b + 1)
        return 0

    lax.fori_loop(0, (nch - 2) // 2, _pair, 0)
    _do_chunk(0, jnp.int32(nch - 1))

    _chunk_dmas(1, nch - 2, start=False)
    _chunk_dmas(0, nch - 1, start=False)
    for s in range(per_w):
        batch = wid * per_w + s
        pltpu.make_async_copy(
            tailb, o_hbm.at[batch, pl.ds(nch * _G, 1)], tsem
        ).wait()


def kernel(x):
    n = x.shape[0]
    inner = x.shape[1:]
    p = int(np.prod(inner))
    hw = inner[-1]
    mesh = plsc.VectorSubcoreMesh(core_axis_name="c", subcore_axis_name="s")
    out = pl.kernel(
        functools.partial(_to_z_sc, n, p, hw),
        out_type=jax.ShapeDtypeStruct((n, 1 + p) + tuple(inner), x.dtype),
        mesh=mesh,
        scratch_types=[
            pltpu.VMEM((2, _G) + tuple(inner), jnp.float32),
            pltpu.VMEM((1,) + tuple(inner), jnp.float32),
            pltpu.SemaphoreType.DMA((2,)),
            pltpu.SemaphoreType.DMA,
        ],
        compiler_params=pltpu.CompilerParams(use_tc_tiling_on_sc=True),
    )(x)
    return out
